# baseline (speedup 1.0000x reference)
import functools

import jax
import jax.numpy as jnp
from jax import lax
from jax.experimental import pallas as pl
from jax.experimental.pallas import tpu as pltpu
from jax.experimental.pallas import tpu_sc as plsc

VOCAB = 1028
C = 9
H = 2048
NC = 2
NS = 16
L = 16
NW = NC * NS

P = 2
GROUP = C * P
STRIDE = 24


def _build_sc_kernel(n_pos: int):
    per_w = n_pos // NW
    items = per_w // P

    mesh = plsc.VectorSubcoreMesh(core_axis_name="c", subcore_axis_name="s")

    @functools.partial(
        pl.kernel,
        mesh=mesh,
        out_type=jax.ShapeDtypeStruct((n_pos, H), jnp.float32),
        scratch_types=[
            pltpu.VMEM((items * STRIDE,), jnp.int32),
            pltpu.VMEM((STRIDE, H), jnp.float32),
            pltpu.VMEM((STRIDE, H), jnp.float32),
            pltpu.VMEM((P, H), jnp.float32),
            pltpu.VMEM((P, H), jnp.float32),
            pltpu.SemaphoreType.DMA,
            pltpu.SemaphoreType.DMA,
            pltpu.SemaphoreType.DMA,
            pltpu.SemaphoreType.DMA,
        ],
    )
    def k(idx_hbm, table_hbm, out_hbm, idx_v, rows0, rows1,
          stage0, stage1, gsem0, gsem1, ssem0, ssem1):
        wid = lax.axis_index("s") * NC + lax.axis_index("c")
        base = wid * per_w
        pltpu.sync_copy(idx_hbm.at[pl.ds(wid * items * STRIDE, items * STRIDE)],
                        idx_v)

        def fire_gather(t, rows, sem):
            pltpu.async_copy(
                table_hbm.at[idx_v.at[pl.ds(t * STRIDE, STRIDE)]], rows, sem)

        def wait_gather(rows, sem):
            pltpu.make_async_copy(
                table_hbm.at[idx_v.at[pl.ds(0, STRIDE)]], rows, sem).wait()

        def fire_store(t, stage, sem):
            pltpu.async_copy(stage, out_hbm.at[pl.ds(base + t * P, P)], sem)

        def wait_store(stage, sem):
            pltpu.make_async_copy(
                stage, out_hbm.at[pl.ds(0, P)], sem).wait()

        def tree(vs):
            while len(vs) > 1:
                nxt = [vs[i] + vs[i + 1] for i in range(0, len(vs) - 1, 2)]
                if len(vs) % 2:
                    nxt.append(vs[-1])
                vs = nxt
            return vs[0]

        def compute(rows, stage):
            for r in range(P):
                @pl.loop(0, H, step=4 * L)
                def _(j):
                    for jj in range(0, 4 * L, L):
                        sl = pl.ds(j + jj, L)
                        stage[r, sl] = tree(
                            [rows[r * C + c, sl] for c in range(C)])

        fire_gather(0, rows0, gsem0)

        @pl.loop(0, items // 2)
        def _(k2):
            t0 = 2 * k2
            wait_gather(rows0, gsem0)
            fire_gather(t0 + 1, rows1, gsem1)

            @pl.when(k2 > 0)
            def _():
                wait_store(stage0, ssem0)
            compute(rows0, stage0)
            fire_store(t0, stage0, ssem0)

            @pl.when(k2 < items // 2 - 1)
            def _():
                fire_gather(t0 + 2, rows0, gsem0)
            wait_gather(rows1, gsem1)

            @pl.when(k2 > 0)
            def _():
                wait_store(stage1, ssem1)
            compute(rows1, stage1)
            fire_store(t0 + 1, stage1, ssem1)

        wait_store(stage0, ssem0)
        wait_store(stage1, ssem1)

    return k


def kernel(audio_codes, embed_table):
    b, s, _ = audio_codes.shape
    n_pos = b * s
    offs = jnp.arange(C, dtype=jnp.int32) * VOCAB
    tok = audio_codes.astype(jnp.int32).reshape(n_pos, C) + offs
    idx = jnp.pad(tok.reshape(n_pos // P, GROUP),
                  ((0, 0), (0, STRIDE - GROUP))).reshape(-1)
    out = _build_sc_kernel(n_pos)(idx, embed_table)
    return out.reshape(b, s, H)


# speedup vs baseline: 1.7464x; 1.7464x over previous
import functools

import jax
import jax.numpy as jnp
from jax import lax
from jax.experimental import pallas as pl
from jax.experimental.pallas import tpu as pltpu
from jax.experimental.pallas import tpu_sc as plsc

VOCAB = 1028
C = 9
H = 2048
NC = 2
NS = 16
L = 16
NW = NC * NS

P = 8
CG = 3
GROUP = CG * P


def _build_sc_kernel(n_pos: int):
    per_w = n_pos // NW
    n_groups = per_w // P
    items = 3 * n_groups
    part = n_pos * CG
    blk = n_groups * GROUP

    mesh = plsc.VectorSubcoreMesh(core_axis_name="c", subcore_axis_name="s")

    @functools.partial(
        pl.kernel,
        mesh=mesh,
        out_type=jax.ShapeDtypeStruct((n_pos, H), jnp.float32),
        scratch_types=[
            pltpu.VMEM((3 * blk,), jnp.int32),
            pltpu.VMEM((GROUP, H), jnp.float32),
            pltpu.VMEM((GROUP, H), jnp.float32),
            pltpu.VMEM((P, H), jnp.float32),
            pltpu.SemaphoreType.DMA,
            pltpu.SemaphoreType.DMA,
            pltpu.SemaphoreType.DMA,
        ],
    )
    def k(idx_hbm, table_hbm, out_hbm, idx_v, rows0, rows1, stage,
          gsem0, gsem1, ssem):
        wid = lax.axis_index("s") * NC + lax.axis_index("c")
        base = wid * per_w
        for cg in range(3):
            pltpu.sync_copy(idx_hbm.at[pl.ds(cg * part + wid * blk, blk)],
                            idx_v.at[pl.ds(cg * blk, blk)])

        rows = (rows0, rows1)
        gsems = (gsem0, gsem1)

        def item_off(k2, m):
            gl = (1 if m >= 3 else 0)
            cg = m % 3
            return 48 * k2 + cg * blk + gl * GROUP

        def fire_gather(off, buf, sem):
            pltpu.async_copy(
                table_hbm.at[idx_v.at[pl.ds(off, GROUP)]], buf, sem)

        def wait_gather(buf, sem):
            pltpu.make_async_copy(
                table_hbm.at[idx_v.at[pl.ds(0, GROUP)]], buf, sem).wait()

        def wait_store():
            pltpu.make_async_copy(
                stage, out_hbm.at[pl.ds(0, P)], ssem).wait()

        def compute(buf, cg):
            for r in range(P):
                @pl.loop(0, H, step=2 * L)
                def _(j):
                    for jj in (0, L):
                        sl = pl.ds(j + jj, L)
                        a = buf[r * CG, sl]
                        b = buf[r * CG + 1, sl]
                        c3 = buf[r * CG + 2, sl]
                        v = (a + b) + c3
                        if cg == 0:
                            stage[r, sl] = v
                        else:
                            stage[r, sl] = stage[r, sl] + v

        fire_gather(0, rows0, gsem0)

        @pl.loop(0, n_groups // 2)
        def _(k2):
            for m in range(6):
                u = 6 * k2 + m
                b = m & 1
                wait_gather(rows[b], gsems[b])

                @pl.when(u < items - 1)
                def _():
                    fire_gather(item_off(k2, m + 1) if m < 5
                                else 48 * (k2 + 1), rows[1 - b], gsems[1 - b])
                cg = m % 3
                if cg == 0:
                    if m == 0:
                        @pl.when(k2 > 0)
                        def _():
                            wait_store()
                    else:
                        wait_store()
                compute(rows[b], cg)
                if cg == 2:
                    g = 2 * k2 + (1 if m >= 3 else 0)
                    pltpu.async_copy(
                        stage, out_hbm.at[pl.ds(base + g * P, P)], ssem)

        wait_store()

    return k


def kernel(audio_codes, embed_table):
    b, s, _ = audio_codes.shape
    n_pos = b * s
    offs = jnp.arange(C, dtype=jnp.int32) * VOCAB
    tok = audio_codes.astype(jnp.int32).reshape(n_pos, C) + offs
    idx = jnp.concatenate(
        [tok[:, 3 * cg:3 * cg + CG].reshape(-1) for cg in range(3)])
    out = _build_sc_kernel(n_pos)(idx, embed_table)
    return out.reshape(b, s, H)


# speedup vs baseline: 3.1943x; 1.8291x over previous
import functools

import jax
import jax.numpy as jnp
from jax import lax
from jax.experimental import pallas as pl
from jax.experimental.pallas import tpu as pltpu
from jax.experimental.pallas import tpu_sc as plsc

VOCAB = 1028
C = 9
H = 2048
NC = 2
NS = 16
L = 16
NW = NC * NS

P = 8
GROUP = C * P
Q = 4
W = H // Q


def _build_sc_kernel(n_pos: int):
    per_w = n_pos // NW
    n_groups = per_w // P

    mesh = plsc.VectorSubcoreMesh(core_axis_name="c", subcore_axis_name="s")

    @functools.partial(
        pl.kernel,
        mesh=mesh,
        out_type=jax.ShapeDtypeStruct((n_pos, H), jnp.float32),
        scratch_types=[
            pltpu.VMEM((n_groups * GROUP,), jnp.int32),
            pltpu.VMEM((GROUP, W), jnp.float32),
            pltpu.VMEM((GROUP, W), jnp.float32),
            pltpu.VMEM((P, W), jnp.float32),
            pltpu.VMEM((P, W), jnp.float32),
            pltpu.SemaphoreType.DMA,
            pltpu.SemaphoreType.DMA,
            pltpu.SemaphoreType.DMA,
            pltpu.SemaphoreType.DMA,
        ],
    )
    def k(idx_hbm, table_hbm, out_hbm, idx_v, rows0, rows1,
          stage0, stage1, gsem0, gsem1, ssem0, ssem1):
        wid = lax.axis_index("s") * NC + lax.axis_index("c")
        base = wid * per_w
        pltpu.sync_copy(
            idx_hbm.at[pl.ds(wid * n_groups * GROUP, n_groups * GROUP)],
            idx_v)

        def fire_gather(t, rows, sem):
            g = lax.bitwise_and(t, n_groups - 1)
            q = lax.shift_right_logical(t, 4)
            pltpu.async_copy(
                table_hbm.at[idx_v.at[pl.ds(g * GROUP, GROUP)],
                             pl.ds(q * W, W)],
                rows, sem)

        def wait_gather(rows, sem):
            pltpu.make_async_copy(
                table_hbm.at[idx_v.at[pl.ds(0, GROUP)], pl.ds(0, W)],
                rows, sem).wait()

        def out_slice(t):
            g = lax.bitwise_and(t, n_groups - 1)
            q = lax.shift_right_logical(t, 4)
            return out_hbm.at[pl.ds(base + g * P, P), pl.ds(q * W, W)]

        def wait_store(stage, sem):
            pltpu.make_async_copy(
                stage, out_hbm.at[pl.ds(0, P), pl.ds(0, W)], sem).wait()

        def tree(vs):
            while len(vs) > 1:
                nxt = [vs[i] + vs[i + 1] for i in range(0, len(vs) - 1, 2)]
                if len(vs) % 2:
                    nxt.append(vs[-1])
                vs = nxt
            return vs[0]

        def compute(rows, stage):
            for r in range(P):
                @pl.loop(0, W, step=2 * L)
                def _(j):
                    for jj in (0, L):
                        sl = pl.ds(j + jj, L)
                        stage[r, sl] = tree(
                            [rows[r * C + c, sl] for c in range(C)])

        n_items = Q * n_groups

        fire_gather(0, rows0, gsem0)

        @pl.loop(0, n_items // 2)
        def _(k2):
            t0 = 2 * k2
            wait_gather(rows0, gsem0)
            fire_gather(t0 + 1, rows1, gsem1)

            @pl.when(k2 > 0)
            def _():
                wait_store(stage0, ssem0)
            compute(rows0, stage0)
            pltpu.async_copy(stage0, out_slice(t0), ssem0)

            @pl.when(k2 < n_items // 2 - 1)
            def _():
                fire_gather(t0 + 2, rows0, gsem0)
            wait_gather(rows1, gsem1)

            @pl.when(k2 > 0)
            def _():
                wait_store(stage1, ssem1)
            compute(rows1, stage1)
            pltpu.async_copy(stage1, out_slice(t0 + 1), ssem1)

        wait_store(stage0, ssem0)
        wait_store(stage1, ssem1)

    return k


def kernel(audio_codes, embed_table):
    b, s, _ = audio_codes.shape
    n_pos = b * s
    offs = jnp.arange(C, dtype=jnp.int32) * VOCAB
    tok = audio_codes.astype(jnp.int32).reshape(n_pos, C) + offs
    idx = tok.reshape(-1)
    out = _build_sc_kernel(n_pos)(idx, embed_table)
    return out.reshape(b, s, H)


# speedup vs baseline: 3.5551x; 1.1129x over previous
import functools

import jax
import jax.numpy as jnp
from jax import lax
from jax.experimental import pallas as pl
from jax.experimental.pallas import tpu as pltpu
from jax.experimental.pallas import tpu_sc as plsc

VOCAB = 1028
C = 9
H = 2048
NC = 2
NS = 16
L = 16
NW = NC * NS

P = 8
GROUP = C * P
HW = H // 2
Q = 2
W = HW // Q


def _build_sc_kernel(n_pos: int):
    per_w = n_pos // NW
    n_groups = per_w // P

    mesh = plsc.VectorSubcoreMesh(core_axis_name="c", subcore_axis_name="s")

    @functools.partial(
        pl.kernel,
        mesh=mesh,
        out_type=jax.ShapeDtypeStruct((n_pos, H), jnp.float32),
        scratch_types=[
            pltpu.VMEM((n_groups * GROUP,), jnp.int32),
            pltpu.VMEM((GROUP, W), jnp.int32),
            pltpu.VMEM((GROUP, W), jnp.int32),
            pltpu.VMEM((2, P, W), jnp.float32),
            pltpu.VMEM((2, P, W), jnp.float32),
            pltpu.SemaphoreType.DMA,
            pltpu.SemaphoreType.DMA,
            pltpu.SemaphoreType.DMA,
            pltpu.SemaphoreType.DMA,
        ],
    )
    def k(idx_hbm, table_hbm, out_hbm, idx_v, rows0, rows1,
          stage0, stage1, gsem0, gsem1, ssem0, ssem1):
        wid = lax.axis_index("s") * NC + lax.axis_index("c")
        base = wid * per_w
        pltpu.sync_copy(
            idx_hbm.at[pl.ds(wid * n_groups * GROUP, n_groups * GROUP)],
            idx_v)

        def fire_gather(t, rows, sem):
            g = lax.bitwise_and(t, n_groups - 1)
            q = lax.shift_right_logical(t, 4)
            pltpu.async_copy(
                table_hbm.at[idx_v.at[pl.ds(g * GROUP, GROUP)],
                             pl.ds(q * W, W)],
                rows, sem)

        def wait_gather(rows, sem):
            pltpu.make_async_copy(
                table_hbm.at[idx_v.at[pl.ds(0, GROUP)], pl.ds(0, W)],
                rows, sem).wait()

        def fire_stores(t, stage, sem):
            g = lax.bitwise_and(t, n_groups - 1)
            q = lax.shift_right_logical(t, 4)
            rsl = pl.ds(base + g * P, P)
            pltpu.async_copy(
                stage.at[0], out_hbm.at[rsl, pl.ds(q * W, W)], sem)
            pltpu.async_copy(
                stage.at[1], out_hbm.at[rsl, pl.ds(HW + q * W, W)], sem)

        def wait_store(stage, sem):
            dummy = out_hbm.at[pl.ds(0, P), pl.ds(0, W)]
            pltpu.make_async_copy(stage.at[0], dummy, sem).wait()
            pltpu.make_async_copy(stage.at[1], dummy, sem).wait()

        def tree(vs):
            while len(vs) > 1:
                nxt = [vs[i] + vs[i + 1] for i in range(0, len(vs) - 1, 2)]
                if len(vs) % 2:
                    nxt.append(vs[-1])
                vs = nxt
            return vs[0]

        himask = jnp.int32(-65536)

        def compute(rows, stage):
            for r in range(P):
                @pl.loop(0, W, step=2 * L)
                def _(j):
                    for jj in (0, L):
                        sl = pl.ds(j + jj, L)
                        los, his = [], []
                        for c in range(C):
                            v = rows[r * C + c, sl]
                            los.append(lax.bitcast_convert_type(
                                v << 16, jnp.float32))
                            his.append(lax.bitcast_convert_type(
                                v & himask, jnp.float32))
                        stage[0, r, sl] = tree(los)
                        stage[1, r, sl] = tree(his)

        n_items = Q * n_groups

        fire_gather(0, rows0, gsem0)

        @pl.loop(0, n_items // 2)
        def _(k2):
            t0 = 2 * k2
            wait_gather(rows0, gsem0)
            fire_gather(t0 + 1, rows1, gsem1)

            @pl.when(k2 > 0)
            def _():
                wait_store(stage0, ssem0)
            compute(rows0, stage0)
            fire_stores(t0, stage0, ssem0)

            @pl.when(k2 < n_items // 2 - 1)
            def _():
                fire_gather(t0 + 2, rows0, gsem0)
            wait_gather(rows1, gsem1)

            @pl.when(k2 > 0)
            def _():
                wait_store(stage1, ssem1)
            compute(rows1, stage1)
            fire_stores(t0 + 1, stage1, ssem1)

        wait_store(stage0, ssem0)
        wait_store(stage1, ssem1)

    return k


def kernel(audio_codes, embed_table):
    b, s, _ = audio_codes.shape
    n_pos = b * s
    offs = jnp.arange(C, dtype=jnp.int32) * VOCAB
    tok = audio_codes.astype(jnp.int32).reshape(n_pos, C) + offs
    idx = tok.reshape(-1)
    lo = lax.bitcast_convert_type(
        embed_table[:, :HW].astype(jnp.bfloat16), jnp.uint16)
    hi = lax.bitcast_convert_type(
        embed_table[:, HW:].astype(jnp.bfloat16), jnp.uint16)
    packed = (hi.astype(jnp.uint32) << 16) | lo.astype(jnp.uint32)
    table_i32 = lax.bitcast_convert_type(packed, jnp.int32)
    out = _build_sc_kernel(n_pos)(idx, table_i32)
    return out.reshape(b, s, H)
